# baseline (speedup 1.0000x reference)
import functools

import jax
import jax.numpy as jnp
from jax import lax
from jax.experimental import pallas as pl
from jax.experimental.pallas import tpu as pltpu
from jax.experimental.pallas import tpu_sc as plsc

VOCAB = 100000
D = 128
B = 1024
L = 200
EPS = 1e-12

NC = 2
NS = 16
NW = NC * NS
N_TOK = B * L
NBUF = 2

K_PIPE = 2
TOK_PER_CALL = N_TOK // K_PIPE
TOK_PER_W = TOK_PER_CALL // NW
CH = 128
N_CHUNKS = TOK_PER_W // CH
assert TOK_PER_W % CH == 0 and TOK_PER_CALL % L == 0


@functools.cache
def _make_sc_gather():
    mesh = plsc.VectorSubcoreMesh(
        core_axis_name="c", subcore_axis_name="s", num_cores=NC, num_subcores=NS
    )
    return functools.partial(
        pl.kernel,
        out_type=jax.ShapeDtypeStruct((TOK_PER_CALL, D), jnp.float32),
        mesh=mesh,
        scratch_types=[
            pltpu.VMEM((N_CHUNKS, CH), jnp.int32),
            pltpu.VMEM((NBUF, CH, D), jnp.float32),
            pltpu.SemaphoreType.DMA,
            pltpu.SemaphoreType.DMA,
        ],
    )(_sc_gather_body)


def _sc_gather_body(ids_hbm, table_hbm, out_hbm, idx_v, rows_v, sem0, sem1):
    wid = lax.axis_index("s") * NC + lax.axis_index("c")
    out_base = wid * TOK_PER_W
    sems = (sem0, sem1)
    pltpu.sync_copy(ids_hbm.at[wid], idx_v)

    def start(chunk, buf):
        return pltpu.async_copy(
            table_hbm.at[idx_v.at[chunk]], rows_v.at[buf], sems[buf]
        )

    for b in range(NBUF):
        start(b, b)

    def body(c, carry):
        for b in range(NBUF):
            chunk = c + b
            pltpu.make_async_copy(
                table_hbm.at[idx_v.at[chunk]], rows_v.at[b], sems[b]
            ).wait()
            pltpu.sync_copy(
                rows_v.at[b], out_hbm.at[pl.ds(out_base + chunk * CH, CH)]
            )

            @pl.when(chunk + NBUF < N_CHUNKS)
            def _():
                start(chunk + NBUF, b)

        return carry

    lax.fori_loop(0, N_CHUNKS // NBUF, lambda i, cy: body(i * NBUF, cy), 0,
                  unroll=False)
    for chunk in range(N_CHUNKS - (N_CHUNKS % NBUF), N_CHUNKS):
        b = chunk % NBUF
        pltpu.make_async_copy(
            table_hbm.at[idx_v.at[chunk]], rows_v.at[b], sems[b]
        ).wait()
        pltpu.sync_copy(
            rows_v.at[b], out_hbm.at[pl.ds(out_base + chunk * CH, CH)]
        )


def _tc_posln_body_first(x_ref, w_ref, b_ref, o_ref, pos_scr):
    _tc_posln_body(x_ref, w_ref, b_ref, o_ref, pos_scr)


def _tc_posln_body_chained(prev_ref, x_ref, w_ref, b_ref, o_ref, pos_scr):
    del prev_ref
    _tc_posln_body(x_ref, w_ref, b_ref, o_ref, pos_scr)


def _tc_posln_body(x_ref, w_ref, b_ref, o_ref, pos_scr):
    @pl.when(pl.program_id(0) == 0)
    def _():
        l_idx = lax.broadcasted_iota(jnp.int32, (L, D // 2), 0).astype(jnp.float32)
        j_idx = lax.broadcasted_iota(jnp.int32, (L, D // 2), 1).astype(jnp.float32)
        inv_freq = jnp.exp(j_idx * (-2.0 / D * jnp.log(10000.0)))
        angle = (L - 1.0 - l_idx) * inv_freq
        pos_scr[...] = jnp.concatenate(
            [jnp.sin(angle), jnp.cos(angle)], axis=-1
        )

    x = x_ref[...]
    e = x + pos_scr[...][None, :, :]
    u = jnp.mean(e, axis=-1, keepdims=True)
    d = e - u
    s = jnp.mean(d * d, axis=-1, keepdims=True)
    y = d * lax.rsqrt(s + EPS)
    o_ref[...] = y * w_ref[...][None, None, :] + b_ref[...][None, None, :]


def _tc_posln_chunk(x, prev, c, ln_weight, ln_bias, S=64):
    nblk = (B // K_PIPE) // S
    base = c * nblk
    x_spec = pl.BlockSpec((S, L, D), lambda i: (i, 0, 0))
    wb_spec = pl.BlockSpec((D,), lambda i: (0,))
    out_spec = pl.BlockSpec((S, L, D), lambda i, _b=base: (i + _b, 0, 0))
    common = dict(
        grid=(nblk,),
        out_specs=out_spec,
        out_shape=jax.ShapeDtypeStruct((B, L, D), jnp.float32),
        scratch_shapes=[pltpu.VMEM((L, D), jnp.float32)],
    )
    if prev is None:
        return pl.pallas_call(
            _tc_posln_body_first,
            in_specs=[x_spec, wb_spec, wb_spec],
            **common,
        )(x, ln_weight, ln_bias)
    return pl.pallas_call(
        _tc_posln_body_chained,
        in_specs=[pl.BlockSpec(memory_space=pl.ANY), x_spec, wb_spec, wb_spec],
        input_output_aliases={0: 0},
        **common,
    )(prev, x, ln_weight, ln_bias)


def _kernel_pipelined(input_ids, table, ln_weight, ln_bias):
    ids = input_ids.astype(jnp.int32).reshape(K_PIPE, NW, N_CHUNKS, CH)
    gather = _make_sc_gather()
    out = None
    for c in range(K_PIPE):
        rows = gather(ids[c], table)
        out = _tc_posln_chunk(
            rows.reshape(B // K_PIPE, L, D), out, c, ln_weight, ln_bias
        )
    return out


F_TOK_W = N_TOK // NW
F_CHUNKS = F_TOK_W // CH


def _tc_pos_body(o_ref):
    l_idx = lax.broadcasted_iota(jnp.int32, (L, D // 2), 0).astype(jnp.float32)
    j_idx = lax.broadcasted_iota(jnp.int32, (L, D // 2), 1).astype(jnp.float32)
    inv_freq = jnp.exp(j_idx * (-2.0 / D * jnp.log(10000.0)))
    angle = (L - 1.0 - l_idx) * inv_freq
    o_ref[...] = jnp.concatenate([jnp.sin(angle), jnp.cos(angle)], axis=-1)


def _tc_pos():
    return pl.pallas_call(
        _tc_pos_body,
        out_shape=jax.ShapeDtypeStruct((L, D), jnp.float32),
    )()


@functools.cache
def _make_sc_fused():
    mesh = plsc.VectorSubcoreMesh(
        core_axis_name="c", subcore_axis_name="s", num_cores=NC, num_subcores=NS
    )
    return functools.partial(
        pl.kernel,
        out_type=jax.ShapeDtypeStruct((N_TOK, D), jnp.float32),
        mesh=mesh,
        scratch_types=[
            pltpu.VMEM((F_CHUNKS, CH), jnp.int32),
            pltpu.VMEM((NBUF, CH, D), jnp.float32),
            pltpu.VMEM((L, D), jnp.float32),
            pltpu.VMEM((D,), jnp.float32),
            pltpu.VMEM((D,), jnp.float32),
            pltpu.SemaphoreType.DMA,
            pltpu.SemaphoreType.DMA,
        ],
    )(_sc_fused_body)


def _lane_shuffle(v, perm):
    return lax.gather(
        v,
        perm[:, None],
        lax.GatherDimensionNumbers(
            offset_dims=(), collapsed_slice_dims=(0,), start_index_map=(0,)
        ),
        slice_sizes=(1,),
        mode=lax.GatherScatterMode.PROMISE_IN_BOUNDS,
    )


def _lane_sum(v, perms):
    for p in perms:
        v = v + _lane_shuffle(v, p)
    return v


def _sc_fused_body(ids_hbm, table_hbm, pos_hbm, w_hbm, b_hbm, out_hbm,
                   idx_v, rows_v, pos_v, w_v, b_v, sem0, sem1):
    wid = lax.axis_index("s") * NC + lax.axis_index("c")
    out_base = wid * F_TOK_W
    sems = (sem0, sem1)
    pltpu.sync_copy(ids_hbm.at[wid], idx_v)
    pltpu.sync_copy(pos_hbm, pos_v)
    pltpu.sync_copy(w_hbm, w_v)
    pltpu.sync_copy(b_hbm, b_v)

    def start(chunk, buf):
        pltpu.async_copy(table_hbm.at[idx_v.at[chunk]], rows_v.at[buf], sems[buf])

    for bb in range(NBUF):
        start(bb, bb)

    lane = lax.broadcasted_iota(jnp.int32, (16,), 0)
    perms = tuple(lax.bitwise_xor(lane, jnp.int32(k)) for k in (1, 2, 4, 8))

    def process(chunk, buf):
        pltpu.make_async_copy(
            table_hbm.at[idx_v.at[chunk]], rows_v.at[buf], sems[buf]
        ).wait()
        rbuf = rows_v.at[buf]
        base_mod = chunk * CH

        def row_body(r, carry):
            pidx = lax.rem(base_mod + r, L)
            xs = []
            for j in range(8):
                xs.append(
                    rbuf[r, pl.ds(j * 16, 16)] + pos_v[pidx, pl.ds(j * 16, 16)]
                )
            sum_v = ((xs[0] + xs[1]) + (xs[2] + xs[3])) + (
                (xs[4] + xs[5]) + (xs[6] + xs[7])
            )
            sq = [x * x for x in xs]
            ssq_v = ((sq[0] + sq[1]) + (sq[2] + sq[3])) + (
                (sq[4] + sq[5]) + (sq[6] + sq[7])
            )
            u = _lane_sum(sum_v, perms) * (1.0 / D)
            var = _lane_sum(ssq_v, perms) * (1.0 / D) - u * u
            t = var + EPS
            ti = lax.bitcast_convert_type(t, jnp.int32)
            y = lax.bitcast_convert_type(
                jnp.full((16,), 0x5F3759DF, jnp.int32) - (ti >> 1), jnp.float32
            )
            for _ in range(3):
                y = y * (1.5 - 0.5 * t * y * y)
            for j in range(8):
                w = w_v[pl.ds(j * 16, 16)]
                bv = b_v[pl.ds(j * 16, 16)]
                rbuf[r, pl.ds(j * 16, 16)] = (xs[j] - u) * (w * y) + bv
            return carry

        lax.fori_loop(0, CH, row_body, 0, unroll=2)
        pltpu.sync_copy(
            rbuf, out_hbm.at[pl.ds(out_base + chunk * CH, CH)]
        )

    def body(c, carry):
        for bb in range(NBUF):
            chunk = c + bb
            process(chunk, bb)

            @pl.when(chunk + NBUF < F_CHUNKS)
            def _():
                start(chunk + NBUF, bb)

        return carry

    lax.fori_loop(0, F_CHUNKS // NBUF, lambda i, cy: body(i * NBUF, cy), 0,
                  unroll=False)


def kernel(input_ids, table, ln_weight, ln_bias):
    pos = _tc_pos()
    ids = input_ids.astype(jnp.int32).reshape(NW, F_CHUNKS, CH)
    out = _make_sc_fused()(ids, table, pos, ln_weight, ln_bias)
    return out.reshape(B, L, D)


# speedup vs baseline: 1.0319x; 1.0319x over previous
import functools

import jax
import jax.numpy as jnp
from jax import lax
from jax.experimental import pallas as pl
from jax.experimental.pallas import tpu as pltpu
from jax.experimental.pallas import tpu_sc as plsc

VOCAB = 100000
D = 128
B = 1024
L = 200
EPS = 1e-12

NC = 2
NS = 16
NW = NC * NS
N_TOK = B * L
NBUF = 2

K_PIPE = 2
TOK_PER_CALL = N_TOK // K_PIPE
TOK_PER_W = TOK_PER_CALL // NW
CH = 128
N_CHUNKS = TOK_PER_W // CH
assert TOK_PER_W % CH == 0 and TOK_PER_CALL % L == 0


@functools.cache
def _make_sc_gather():
    mesh = plsc.VectorSubcoreMesh(
        core_axis_name="c", subcore_axis_name="s", num_cores=NC, num_subcores=NS
    )
    return functools.partial(
        pl.kernel,
        out_type=jax.ShapeDtypeStruct((TOK_PER_CALL, D), jnp.float32),
        mesh=mesh,
        scratch_types=[
            pltpu.VMEM((N_CHUNKS, CH), jnp.int32),
            pltpu.VMEM((NBUF, CH, D), jnp.float32),
            pltpu.SemaphoreType.DMA,
            pltpu.SemaphoreType.DMA,
        ],
    )(_sc_gather_body)


def _sc_gather_body(ids_hbm, table_hbm, out_hbm, idx_v, rows_v, sem0, sem1):
    wid = lax.axis_index("s") * NC + lax.axis_index("c")
    out_base = wid * TOK_PER_W
    sems = (sem0, sem1)
    pltpu.sync_copy(ids_hbm.at[wid], idx_v)

    def start(chunk, buf):
        return pltpu.async_copy(
            table_hbm.at[idx_v.at[chunk]], rows_v.at[buf], sems[buf]
        )

    for b in range(NBUF):
        start(b, b)

    def body(c, carry):
        for b in range(NBUF):
            chunk = c + b
            pltpu.make_async_copy(
                table_hbm.at[idx_v.at[chunk]], rows_v.at[b], sems[b]
            ).wait()
            pltpu.sync_copy(
                rows_v.at[b], out_hbm.at[pl.ds(out_base + chunk * CH, CH)]
            )

            @pl.when(chunk + NBUF < N_CHUNKS)
            def _():
                start(chunk + NBUF, b)

        return carry

    lax.fori_loop(0, N_CHUNKS // NBUF, lambda i, cy: body(i * NBUF, cy), 0,
                  unroll=False)
    for chunk in range(N_CHUNKS - (N_CHUNKS % NBUF), N_CHUNKS):
        b = chunk % NBUF
        pltpu.make_async_copy(
            table_hbm.at[idx_v.at[chunk]], rows_v.at[b], sems[b]
        ).wait()
        pltpu.sync_copy(
            rows_v.at[b], out_hbm.at[pl.ds(out_base + chunk * CH, CH)]
        )


def _tc_posln_body_first(x_ref, w_ref, b_ref, o_ref, pos_scr):
    _tc_posln_body(x_ref, w_ref, b_ref, o_ref, pos_scr)


def _tc_posln_body_chained(prev_ref, x_ref, w_ref, b_ref, o_ref, pos_scr):
    del prev_ref
    _tc_posln_body(x_ref, w_ref, b_ref, o_ref, pos_scr)


def _tc_posln_body(x_ref, w_ref, b_ref, o_ref, pos_scr):
    @pl.when(pl.program_id(0) == 0)
    def _():
        l_idx = lax.broadcasted_iota(jnp.int32, (L, D // 2), 0).astype(jnp.float32)
        j_idx = lax.broadcasted_iota(jnp.int32, (L, D // 2), 1).astype(jnp.float32)
        inv_freq = jnp.exp(j_idx * (-2.0 / D * jnp.log(10000.0)))
        angle = (L - 1.0 - l_idx) * inv_freq
        pos_scr[...] = jnp.concatenate(
            [jnp.sin(angle), jnp.cos(angle)], axis=-1
        )

    x = x_ref[...]
    e = x + pos_scr[...][None, :, :]
    u = jnp.mean(e, axis=-1, keepdims=True)
    d = e - u
    s = jnp.mean(d * d, axis=-1, keepdims=True)
    y = d * lax.rsqrt(s + EPS)
    o_ref[...] = y * w_ref[...][None, None, :] + b_ref[...][None, None, :]


def _tc_posln_chunk(x, prev, c, ln_weight, ln_bias, S=64):
    nblk = (B // K_PIPE) // S
    base = c * nblk
    x_spec = pl.BlockSpec((S, L, D), lambda i: (i, 0, 0))
    wb_spec = pl.BlockSpec((D,), lambda i: (0,))
    out_spec = pl.BlockSpec((S, L, D), lambda i, _b=base: (i + _b, 0, 0))
    common = dict(
        grid=(nblk,),
        out_specs=out_spec,
        out_shape=jax.ShapeDtypeStruct((B, L, D), jnp.float32),
        scratch_shapes=[pltpu.VMEM((L, D), jnp.float32)],
    )
    if prev is None:
        return pl.pallas_call(
            _tc_posln_body_first,
            in_specs=[x_spec, wb_spec, wb_spec],
            **common,
        )(x, ln_weight, ln_bias)
    return pl.pallas_call(
        _tc_posln_body_chained,
        in_specs=[pl.BlockSpec(memory_space=pl.ANY), x_spec, wb_spec, wb_spec],
        input_output_aliases={0: 0},
        **common,
    )(prev, x, ln_weight, ln_bias)


def _kernel_pipelined(input_ids, table, ln_weight, ln_bias):
    ids = input_ids.astype(jnp.int32).reshape(K_PIPE, NW, N_CHUNKS, CH)
    gather = _make_sc_gather()
    out = None
    for c in range(K_PIPE):
        rows = gather(ids[c], table)
        out = _tc_posln_chunk(
            rows.reshape(B // K_PIPE, L, D), out, c, ln_weight, ln_bias
        )
    return out


F_TOK_W = N_TOK // NW
F_CHUNKS = F_TOK_W // CH


def _tc_pos_body(o_ref):
    l_idx = lax.broadcasted_iota(jnp.int32, (L, D // 2), 0).astype(jnp.float32)
    j_idx = lax.broadcasted_iota(jnp.int32, (L, D // 2), 1).astype(jnp.float32)
    inv_freq = jnp.exp(j_idx * (-2.0 / D * jnp.log(10000.0)))
    angle = (L - 1.0 - l_idx) * inv_freq
    o_ref[...] = jnp.concatenate([jnp.sin(angle), jnp.cos(angle)], axis=-1)


def _tc_pos():
    return pl.pallas_call(
        _tc_pos_body,
        out_shape=jax.ShapeDtypeStruct((L, D), jnp.float32),
    )()


@functools.cache
def _make_sc_fused():
    mesh = plsc.VectorSubcoreMesh(
        core_axis_name="c", subcore_axis_name="s", num_cores=NC, num_subcores=NS
    )
    return functools.partial(
        pl.kernel,
        out_type=jax.ShapeDtypeStruct((N_TOK, D), jnp.float32),
        mesh=mesh,
        scratch_types=[
            pltpu.VMEM((F_CHUNKS, CH), jnp.int32),
            pltpu.VMEM((NBUF, CH, D), jnp.float32),
            pltpu.VMEM((L, D), jnp.float32),
            pltpu.VMEM((D,), jnp.float32),
            pltpu.VMEM((D,), jnp.float32),
            pltpu.SemaphoreType.DMA,
            pltpu.SemaphoreType.DMA,
        ],
    )(_sc_fused_body)


def _lane_shuffle(v, perm):
    return lax.gather(
        v,
        perm[:, None],
        lax.GatherDimensionNumbers(
            offset_dims=(), collapsed_slice_dims=(0,), start_index_map=(0,)
        ),
        slice_sizes=(1,),
        mode=lax.GatherScatterMode.PROMISE_IN_BOUNDS,
    )


def _lane_sum(v, perms):
    for p in perms:
        v = v + _lane_shuffle(v, p)
    return v


def _sc_fused_body(ids_hbm, table_hbm, pos_hbm, w_hbm, b_hbm, out_hbm,
                   idx_v, rows_v, pos_v, w_v, b_v, sem0, sem1):
    wid = lax.axis_index("s") * NC + lax.axis_index("c")
    out_base = wid * F_TOK_W
    sems = (sem0, sem1)
    pltpu.sync_copy(ids_hbm.at[wid], idx_v)
    pltpu.sync_copy(pos_hbm, pos_v)
    pltpu.sync_copy(w_hbm, w_v)
    pltpu.sync_copy(b_hbm, b_v)

    def start(chunk, buf):
        pltpu.async_copy(table_hbm.at[idx_v.at[chunk]], rows_v.at[buf], sems[buf])

    for bb in range(NBUF):
        start(bb, bb)

    lane = lax.broadcasted_iota(jnp.int32, (16,), 0)
    perms = tuple(lax.bitwise_xor(lane, jnp.int32(k)) for k in (1, 2, 4, 8))

    def process(chunk, buf):
        pltpu.make_async_copy(
            table_hbm.at[idx_v.at[chunk]], rows_v.at[buf], sems[buf]
        ).wait()
        rbuf = rows_v.at[buf]
        base_mod = chunk * CH

        def row_body(r, carry):
            pidx = lax.rem(base_mod + r, L)
            xs = []
            for j in range(8):
                xs.append(
                    rbuf[r, pl.ds(j * 16, 16)] + pos_v[pidx, pl.ds(j * 16, 16)]
                )
            sum_v = ((xs[0] + xs[1]) + (xs[2] + xs[3])) + (
                (xs[4] + xs[5]) + (xs[6] + xs[7])
            )
            sq = [x * x for x in xs]
            ssq_v = ((sq[0] + sq[1]) + (sq[2] + sq[3])) + (
                (sq[4] + sq[5]) + (sq[6] + sq[7])
            )
            u = _lane_sum(sum_v, perms) * (1.0 / D)
            var = _lane_sum(ssq_v, perms) * (1.0 / D) - u * u
            t = var + EPS
            ti = lax.bitcast_convert_type(t, jnp.int32)
            y = lax.bitcast_convert_type(
                jnp.full((16,), 0x5F3759DF, jnp.int32) - (ti >> 1), jnp.float32
            )
            for _ in range(3):
                y = y * (1.5 - 0.5 * t * y * y)
            for j in range(8):
                w = w_v[pl.ds(j * 16, 16)]
                bv = b_v[pl.ds(j * 16, 16)]
                rbuf[r, pl.ds(j * 16, 16)] = (xs[j] - u) * (w * y) + bv
            return carry

        @plsc.parallel_loop(0, CH, 1, unroll=4)
        def _(r):
            row_body(r, 0)
        pltpu.sync_copy(
            rbuf, out_hbm.at[pl.ds(out_base + chunk * CH, CH)]
        )

    def body(c, carry):
        for bb in range(NBUF):
            chunk = c + bb
            process(chunk, bb)

            @pl.when(chunk + NBUF < F_CHUNKS)
            def _():
                start(chunk + NBUF, bb)

        return carry

    lax.fori_loop(0, F_CHUNKS // NBUF, lambda i, cy: body(i * NBUF, cy), 0,
                  unroll=False)


def kernel(input_ids, table, ln_weight, ln_bias):
    pos = _tc_pos()
    ids = input_ids.astype(jnp.int32).reshape(NW, F_CHUNKS, CH)
    out = _make_sc_fused()(ids, table, pos, ln_weight, ln_bias)
    return out.reshape(B, L, D)


# speedup vs baseline: 3.8345x; 3.7160x over previous
import functools

import jax
import jax.numpy as jnp
from jax import lax
from jax.experimental import pallas as pl
from jax.experimental.pallas import tpu as pltpu
from jax.experimental.pallas import tpu_sc as plsc

VOCAB = 100000
D = 128
B = 1024
L = 200
EPS = 1e-12

NC = 2
NS = 16
NW = NC * NS
NBUF = 2

SPLITS = (128, 448, 448)
CH = 80
S = 64


@functools.cache
def _make_sc_gather(tok_call):
    tok_w = tok_call // NW
    n_chunks = tok_w // CH
    assert tok_w % CH == 0

    def body(ids_hbm, table_hbm, out_hbm, idx_v, rows_v, sem0, sem1):
        wid = lax.axis_index("s") * NC + lax.axis_index("c")
        out_base = wid * tok_w
        sems = (sem0, sem1)
        pltpu.sync_copy(ids_hbm.at[wid], idx_v)

        def start(chunk, buf):
            pltpu.async_copy(
                table_hbm.at[idx_v.at[chunk]], rows_v.at[buf], sems[buf]
            )

        def finish(chunk, buf):
            pltpu.make_async_copy(
                table_hbm.at[idx_v.at[chunk]], rows_v.at[buf], sems[buf]
            ).wait()
            pltpu.sync_copy(
                rows_v.at[buf], out_hbm.at[pl.ds(out_base + chunk * CH, CH)]
            )

        for bb in range(NBUF):
            start(bb, bb)

        def step(c, carry):
            for bb in range(NBUF):
                chunk = c + bb
                finish(chunk, bb)

                @pl.when(chunk + NBUF < n_chunks)
                def _():
                    start(chunk + NBUF, bb)

            return carry

        lax.fori_loop(0, n_chunks // NBUF, lambda i, cy: step(i * NBUF, cy),
                      0, unroll=False)
        for chunk in range(n_chunks - (n_chunks % NBUF), n_chunks):
            finish(chunk, chunk % NBUF)

    mesh = plsc.VectorSubcoreMesh(
        core_axis_name="c", subcore_axis_name="s", num_cores=NC, num_subcores=NS
    )
    return functools.partial(
        pl.kernel,
        out_type=jax.ShapeDtypeStruct((tok_call, D), jnp.float32),
        mesh=mesh,
        scratch_types=[
            pltpu.VMEM((n_chunks, CH), jnp.int32),
            pltpu.VMEM((NBUF, CH, D), jnp.float32),
            pltpu.SemaphoreType.DMA,
            pltpu.SemaphoreType.DMA,
        ],
    )(body)


def _tc_posln_compute(x_ref, w_ref, b_ref, o_ref, pos_scr):
    @pl.when(pl.program_id(0) == 0)
    def _():
        l_idx = lax.broadcasted_iota(jnp.int32, (L, D // 2), 0).astype(jnp.float32)
        j_idx = lax.broadcasted_iota(jnp.int32, (L, D // 2), 1).astype(jnp.float32)
        inv_freq = jnp.exp(j_idx * (-2.0 / D * jnp.log(10000.0)))
        angle = (L - 1.0 - l_idx) * inv_freq
        pos_scr[...] = jnp.concatenate([jnp.sin(angle), jnp.cos(angle)], axis=-1)

    x = x_ref[...]
    e = x + pos_scr[...][None, :, :]
    u = jnp.mean(e, axis=-1, keepdims=True)
    d = e - u
    s = jnp.mean(d * d, axis=-1, keepdims=True)
    y = d * lax.rsqrt(s + EPS)
    o_ref[...] = y * w_ref[...][None, None, :] + b_ref[...][None, None, :]


def _tc_posln_first(x_ref, w_ref, b_ref, o_ref, pos_scr):
    _tc_posln_compute(x_ref, w_ref, b_ref, o_ref, pos_scr)


def _tc_posln_chained(prev_ref, x_ref, w_ref, b_ref, o_ref, pos_scr):
    del prev_ref
    _tc_posln_compute(x_ref, w_ref, b_ref, o_ref, pos_scr)


def _tc_posln_stage(x, prev, seq_base, nseq, ln_weight, ln_bias):
    nblk = nseq // S
    base = seq_base // S
    x_spec = pl.BlockSpec((S, L, D), lambda i: (i, 0, 0))
    wb_spec = pl.BlockSpec((D,), lambda i: (0,))
    out_spec = pl.BlockSpec((S, L, D), lambda i, _b=base: (i + _b, 0, 0))
    common = dict(
        grid=(nblk,),
        out_specs=out_spec,
        out_shape=jax.ShapeDtypeStruct((B, L, D), jnp.float32),
        scratch_shapes=[pltpu.VMEM((L, D), jnp.float32)],
    )
    if prev is None:
        return pl.pallas_call(
            _tc_posln_first,
            in_specs=[x_spec, wb_spec, wb_spec],
            **common,
        )(x, ln_weight, ln_bias)
    return pl.pallas_call(
        _tc_posln_chained,
        in_specs=[pl.BlockSpec(memory_space=pl.ANY), x_spec, wb_spec, wb_spec],
        input_output_aliases={0: 0},
        **common,
    )(prev, x, ln_weight, ln_bias)


def kernel(input_ids, table, ln_weight, ln_bias):
    flat_ids = input_ids.astype(jnp.int32).reshape(-1)
    out = None
    seq_base = 0
    for nseq in SPLITS:
        tok = nseq * L
        ids_c = lax.dynamic_slice(flat_ids, (seq_base * L,), (tok,)).reshape(
            NW, tok // (NW * CH), CH
        )
        rows = _make_sc_gather(tok)(ids_c, table)
        out = _tc_posln_stage(
            rows.reshape(nseq, L, D), out, seq_base, nseq, ln_weight, ln_bias
        )
        seq_base += nseq
    return out


# speedup vs baseline: 3.9726x; 1.0360x over previous
import functools

import jax
import jax.numpy as jnp
from jax import lax
from jax.experimental import pallas as pl
from jax.experimental.pallas import tpu as pltpu
from jax.experimental.pallas import tpu_sc as plsc

VOCAB = 100000
D = 128
B = 1024
L = 200
EPS = 1e-12

NC = 2
NS = 16
NW = NC * NS
NBUF = 2

SPLITS = (512, 512)
CH = 128
S = 64


@functools.cache
def _make_sc_gather(tok_call):
    tok_w = tok_call // NW
    n_chunks = tok_w // CH
    assert tok_w % CH == 0

    def body(ids_hbm, table_hbm, out_hbm, idx_v, rows_v, sem0, sem1):
        wid = lax.axis_index("s") * NC + lax.axis_index("c")
        out_base = wid * tok_w
        sems = (sem0, sem1)
        pltpu.sync_copy(ids_hbm.at[wid], idx_v)

        def start(chunk, buf):
            pltpu.async_copy(
                table_hbm.at[idx_v.at[chunk]], rows_v.at[buf], sems[buf]
            )

        def finish(chunk, buf):
            pltpu.make_async_copy(
                table_hbm.at[idx_v.at[chunk]], rows_v.at[buf], sems[buf]
            ).wait()
            pltpu.sync_copy(
                rows_v.at[buf], out_hbm.at[pl.ds(out_base + chunk * CH, CH)]
            )

        for bb in range(NBUF):
            start(bb, bb)

        def step(c, carry):
            for bb in range(NBUF):
                chunk = c + bb
                finish(chunk, bb)

                @pl.when(chunk + NBUF < n_chunks)
                def _():
                    start(chunk + NBUF, bb)

            return carry

        lax.fori_loop(0, n_chunks // NBUF, lambda i, cy: step(i * NBUF, cy),
                      0, unroll=False)
        for chunk in range(n_chunks - (n_chunks % NBUF), n_chunks):
            finish(chunk, chunk % NBUF)

    mesh = plsc.VectorSubcoreMesh(
        core_axis_name="c", subcore_axis_name="s", num_cores=NC, num_subcores=NS
    )
    return functools.partial(
        pl.kernel,
        out_type=jax.ShapeDtypeStruct((tok_call, D), jnp.float32),
        mesh=mesh,
        scratch_types=[
            pltpu.VMEM((n_chunks, CH), jnp.int32),
            pltpu.VMEM((NBUF, CH, D), jnp.float32),
            pltpu.SemaphoreType.DMA,
            pltpu.SemaphoreType.DMA,
        ],
    )(body)


def _tc_posln_compute(x_ref, w_ref, b_ref, o_ref, pos_scr):
    @pl.when(pl.program_id(0) == 0)
    def _():
        l_idx = lax.broadcasted_iota(jnp.int32, (L, D // 2), 0).astype(jnp.float32)
        j_idx = lax.broadcasted_iota(jnp.int32, (L, D // 2), 1).astype(jnp.float32)
        inv_freq = jnp.exp(j_idx * (-2.0 / D * jnp.log(10000.0)))
        angle = (L - 1.0 - l_idx) * inv_freq
        pos_scr[...] = jnp.concatenate([jnp.sin(angle), jnp.cos(angle)], axis=-1)

    x = x_ref[...]
    e = (x + pos_scr[...][None, :, :]).reshape(S * L, D)
    w_red = jnp.where(
        lax.broadcasted_iota(jnp.int32, (D, 8), 1) == 0, 1.0 / D, 0.0
    )
    u = lax.dot_general(
        e, w_red, (((1,), (0,)), ((), ())), preferred_element_type=jnp.float32
    )[:, 0:1]
    s2 = lax.dot_general(
        e * e, w_red, (((1,), (0,)), ((), ())),
        preferred_element_type=jnp.float32,
    )[:, 0:1]
    var = s2 - u * u
    y = (e - u) * lax.rsqrt(var + EPS)
    o_ref[...] = (
        y * w_ref[...][None, :] + b_ref[...][None, :]
    ).reshape(S, L, D)


def _tc_posln_first(x_ref, w_ref, b_ref, o_ref, pos_scr):
    _tc_posln_compute(x_ref, w_ref, b_ref, o_ref, pos_scr)


def _tc_posln_chained(prev_ref, x_ref, w_ref, b_ref, o_ref, pos_scr):
    del prev_ref
    _tc_posln_compute(x_ref, w_ref, b_ref, o_ref, pos_scr)


def _tc_posln_stage(x, prev, seq_base, nseq, ln_weight, ln_bias):
    nblk = nseq // S
    base = seq_base // S
    x_spec = pl.BlockSpec((S, L, D), lambda i: (i, 0, 0))
    wb_spec = pl.BlockSpec((D,), lambda i: (0,))
    out_spec = pl.BlockSpec((S, L, D), lambda i, _b=base: (i + _b, 0, 0))
    common = dict(
        grid=(nblk,),
        out_specs=out_spec,
        out_shape=jax.ShapeDtypeStruct((B, L, D), jnp.float32),
        scratch_shapes=[pltpu.VMEM((L, D), jnp.float32)],
    )
    if prev is None:
        return pl.pallas_call(
            _tc_posln_first,
            in_specs=[x_spec, wb_spec, wb_spec],
            **common,
        )(x, ln_weight, ln_bias)
    return pl.pallas_call(
        _tc_posln_chained,
        in_specs=[pl.BlockSpec(memory_space=pl.ANY), x_spec, wb_spec, wb_spec],
        input_output_aliases={0: 0},
        **common,
    )(prev, x, ln_weight, ln_bias)


def kernel(input_ids, table, ln_weight, ln_bias):
    flat_ids = input_ids.astype(jnp.int32).reshape(-1)
    out = None
    seq_base = 0
    for nseq in SPLITS:
        tok = nseq * L
        ids_c = lax.dynamic_slice(flat_ids, (seq_base * L,), (tok,)).reshape(
            NW, tok // (NW * CH), CH
        )
        rows = _make_sc_gather(tok)(ids_c, table)
        out = _tc_posln_stage(
            rows.reshape(nseq, L, D), out, seq_base, nseq, ln_weight, ln_bias
        )
        seq_base += nseq
    return out
